# baseline (speedup 1.0000x reference)
import functools

import jax
import jax.numpy as jnp
from jax import lax
from jax.experimental import pallas as pl
from jax.experimental.pallas import tpu as pltpu
from jax.experimental.pallas import tpu_sc as plsc

B = 16
S = 768
H = 768
MAX_SEG = 256
OUT_ROWS_PER_B = MAX_SEG + S
NC, NS = 2, 16
BPC = B // NC
TPT = S // 2
WCH = 64
NWC = TPT // WCH
SCH = 128
NSC = TPT // SCH

_mesh = plsc.VectorSubcoreMesh(core_axis_name="c", subcore_axis_name="s")


@functools.partial(
    pl.kernel,
    mesh=_mesh,
    out_type=jax.ShapeDtypeStruct((B * S, H), jnp.float32),
    scratch_types=[
        pltpu.VMEM((WCH, H), jnp.float32),
        pltpu.VMEM((WCH, H), jnp.float32),
        pltpu.VMEM((TPT,), jnp.int32),
        pltpu.SemaphoreType.DMA,
        pltpu.SemaphoreType.DMA,
        pltpu.SemaphoreType.DMA,
        pltpu.SemaphoreType.DMA,
    ],
)
def _sc_words(ids_f, word_table, wemb, wrows0, wrows1, wid_idx,
              gsem0, gsem1, wsem0, wsem1):
  cid = lax.axis_index("c")
  sid = lax.axis_index("s")
  b = cid * BPC + sid // 2
  half = sid % 2
  tok0 = b * S + half * TPT
  pltpu.sync_copy(ids_f.at[pl.ds(tok0, TPT)], wid_idx)

  wbufs = (wrows0, wrows1)
  gsems = (gsem0, gsem1)
  wsems = (wsem0, wsem1)
  gh = [None] * NWC
  wh = [None] * NWC
  gh[0] = pltpu.async_copy(
      word_table.at[wid_idx.at[pl.ds(0, WCH)]], wbufs[0], gsems[0])
  for c in range(NWC):
    gh[c].wait()
    wh[c] = pltpu.async_copy(
        wbufs[c % 2], wemb.at[pl.ds(tok0 + c * WCH, WCH)], wsems[c % 2])
    if c + 1 < NWC:
      if c >= 1:
        wh[c - 1].wait()
      gh[c + 1] = pltpu.async_copy(
          word_table.at[wid_idx.at[pl.ds((c + 1) * WCH, WCH)]],
          wbufs[(c + 1) % 2], gsems[(c + 1) % 2])
  wh[NWC - 2].wait()
  wh[NWC - 1].wait()


@functools.partial(
    pl.kernel,
    mesh=_mesh,
    out_type=jax.ShapeDtypeStruct((B * OUT_ROWS_PER_B, H), jnp.float32),
    scratch_types=[
        pltpu.VMEM_SHARED((4096, 128), jnp.float32),
        pltpu.VMEM((SCH, 128), jnp.float32),
        pltpu.VMEM((SCH, 128), jnp.float32),
        pltpu.VMEM((6 * TPT,), jnp.int32),
        pltpu.VMEM((TPT,), jnp.int32),
        pltpu.VMEM((TPT,), jnp.int32),
        pltpu.VMEM((TPT,), jnp.int32),
        pltpu.VMEM((TPT,), jnp.int32),
        pltpu.SemaphoreType.DMA,
        pltpu.SemaphoreType.DMA,
        pltpu.SemaphoreType.DMA,
        pltpu.SemaphoreType.DMA,
        pltpu.SemaphoreType.DMA,
    ],
)
def _sc_spatial(x0f, y0f, wf, hf, tables, seg_rows, out, tab_sh, srows0,
                srows1, sp_idx, x0b, y0b, wb, hb, gsem0, gsem1, wsem0, wsem1,
                msem):
  cid = lax.axis_index("c")
  sid = lax.axis_index("s")
  b = cid * BPC + sid // 2
  half = sid % 2
  tok0 = b * S + half * TPT

  seg0 = b * MAX_SEG + half * (MAX_SEG // 2)
  og0 = b * OUT_ROWS_PER_B + half * (MAX_SEG // 2)
  mh = pltpu.async_copy(seg_rows.at[pl.ds(seg0, MAX_SEG // 2)],
                        out.at[pl.ds(og0, MAX_SEG // 2)], msem)

  pltpu.sync_copy(tables.at[pl.ds(sid * 256, 256)],
                  tab_sh.at[pl.ds(sid * 256, 256)])

  l0 = pltpu.async_copy(x0f.at[pl.ds(tok0, TPT)], x0b, gsem0)
  l1 = pltpu.async_copy(y0f.at[pl.ds(tok0, TPT)], y0b, gsem1)
  l2 = pltpu.async_copy(wf.at[pl.ds(tok0, TPT)], wb, wsem0)
  l3 = pltpu.async_copy(hf.at[pl.ds(tok0, TPT)], hb, wsem1)
  l0.wait()
  l1.wait()
  l2.wait()
  l3.wait()
  for j in range(TPT // 16):
    sl = pl.ds(j * 16, 16)
    x0v = x0b[sl]
    y0v = y0b[sl]
    wv = wb[sl]
    hv = hb[sl]
    o = j * 16
    sp_idx[pl.ds(0 * TPT + o, 16)] = x0v
    sp_idx[pl.ds(1 * TPT + o, 16)] = y0v + 1024
    sp_idx[pl.ds(2 * TPT + o, 16)] = x0v + wv
    sp_idx[pl.ds(3 * TPT + o, 16)] = y0v + hv + 1024
    sp_idx[pl.ds(4 * TPT + o, 16)] = hv + 2048
    sp_idx[pl.ds(5 * TPT + o, 16)] = wv + 3072


  sp_out0 = b * OUT_ROWS_PER_B + MAX_SEG + half * TPT
  sbufs = (srows0, srows1)
  gsems = (gsem0, gsem1)
  wsems = (wsem0, wsem1)
  NSP = 6 * NSC
  sgh = [None] * NSP
  swh = [None] * NSP
  def sp_src(i):
    f, c = i // NSC, i % NSC
    return tab_sh.at[sp_idx.at[pl.ds(f * TPT + c * SCH, SCH)]]
  def sp_dst(i):
    f, c = i // NSC, i % NSC
    return out.at[pl.ds(sp_out0 + c * SCH, SCH), pl.ds(f * 128, 128)]
  plsc.subcore_barrier()
  sgh[0] = pltpu.async_copy(sp_src(0), sbufs[0], gsems[0])
  for i in range(NSP):
    sgh[i].wait()
    swh[i] = pltpu.async_copy(sbufs[i % 2], sp_dst(i), wsems[i % 2])
    if i + 1 < NSP:
      if i >= 1:
        swh[i - 1].wait()
      sgh[i + 1] = pltpu.async_copy(sp_src(i + 1), sbufs[(i + 1) % 2],
                                    gsems[(i + 1) % 2])
  swh[NSP - 2].wait()
  swh[NSP - 1].wait()
  mh.wait()


def _tc_seg_body(seg_ref, wemb_ref, out_ref):
  seg = seg_ref[0, 0, :]
  iota_m = lax.broadcasted_iota(jnp.int32, (MAX_SEG, S), 0)
  mask = (iota_m == seg[None, :]).astype(jnp.float32)
  counts = jnp.sum(mask, axis=1, keepdims=True)
  mask = mask / jnp.maximum(counts, 1.0)
  out_ref[0] = jnp.dot(mask, wemb_ref[0],
                       preferred_element_type=jnp.float32)


_tc_seg = pl.pallas_call(
    _tc_seg_body,
    grid=(B,),
    in_specs=[
        pl.BlockSpec((1, 1, S), lambda b: (b, 0, 0)),
        pl.BlockSpec((1, S, H), lambda b: (b, 0, 0)),
    ],
    out_specs=pl.BlockSpec((1, MAX_SEG, H), lambda b: (b, 0, 0)),
    out_shape=jax.ShapeDtypeStruct((B, MAX_SEG, H), jnp.float32),
)


def kernel(input_ids, segment_ids, bbox_xy, bbox_wh,
           word_table, x_table, y_table, h_table, w_table):
  ids_f = input_ids.astype(jnp.int32).reshape(-1)
  x0f = bbox_xy[:, :, 0].astype(jnp.int32).reshape(-1)
  y0f = bbox_xy[:, :, 1].astype(jnp.int32).reshape(-1)
  wf = bbox_wh[:, :, 0].astype(jnp.int32).reshape(-1)
  hf = bbox_wh[:, :, 1].astype(jnp.int32).reshape(-1)
  tables = jnp.concatenate([x_table, y_table, h_table, w_table], axis=0)
  wemb = _sc_words(ids_f, word_table)
  seg3d = segment_ids.astype(jnp.int32).reshape(B, 1, S)
  seg_out = _tc_seg(seg3d, wemb.reshape(B, S, H))
  out = _sc_spatial(x0f, y0f, wf, hf, tables,
                    seg_out.reshape(B * MAX_SEG, H))
  return out.reshape(B, OUT_ROWS_PER_B, H)


# speedup vs baseline: 4.8898x; 4.8898x over previous
import functools

import jax
import jax.numpy as jnp
from jax import lax
from jax.experimental import pallas as pl
from jax.experimental.pallas import tpu as pltpu
from jax.experimental.pallas import tpu_sc as plsc

B = 16
S = 768
H = 768
MAX_SEG = 256
OUT_ROWS_PER_B = MAX_SEG + S
NC, NS = 2, 16
BPC = B // NC
TPT = S // 2
WCH = 64
NWC = TPT // WCH
SCH = 128
NSC = TPT // SCH

_mesh = plsc.VectorSubcoreMesh(core_axis_name="c", subcore_axis_name="s")


@functools.partial(
    pl.kernel,
    mesh=_mesh,
    out_type=jax.ShapeDtypeStruct((B * S, H), jnp.float32),
    scratch_types=[
        pltpu.VMEM((WCH, H), jnp.float32),
        pltpu.VMEM((WCH, H), jnp.float32),
        pltpu.VMEM((TPT,), jnp.int32),
        pltpu.SemaphoreType.DMA,
        pltpu.SemaphoreType.DMA,
        pltpu.SemaphoreType.DMA,
        pltpu.SemaphoreType.DMA,
    ],
)
def _sc_words(ids_f, word_table, wemb, wrows0, wrows1, wid_idx,
              gsem0, gsem1, wsem0, wsem1):
  cid = lax.axis_index("c")
  sid = lax.axis_index("s")
  b = cid * BPC + sid // 2
  half = sid % 2
  tok0 = b * S + half * TPT
  pltpu.sync_copy(ids_f.at[pl.ds(tok0, TPT)], wid_idx)

  wbufs = (wrows0, wrows1)
  gsems = (gsem0, gsem1)
  wsems = (wsem0, wsem1)
  gh = [None] * NWC
  wh = [None] * NWC
  gh[0] = pltpu.async_copy(
      word_table.at[wid_idx.at[pl.ds(0, WCH)]], wbufs[0], gsems[0])
  for c in range(NWC):
    gh[c].wait()
    wh[c] = pltpu.async_copy(
        wbufs[c % 2], wemb.at[pl.ds(tok0 + c * WCH, WCH)], wsems[c % 2])
    if c + 1 < NWC:
      if c >= 1:
        wh[c - 1].wait()
      gh[c + 1] = pltpu.async_copy(
          word_table.at[wid_idx.at[pl.ds((c + 1) * WCH, WCH)]],
          wbufs[(c + 1) % 2], gsems[(c + 1) % 2])
  wh[NWC - 2].wait()
  wh[NWC - 1].wait()


@functools.partial(
    pl.kernel,
    mesh=_mesh,
    out_type=jax.ShapeDtypeStruct((B * OUT_ROWS_PER_B, H), jnp.float32),
    scratch_types=[
        pltpu.VMEM_SHARED((4096, 128), jnp.float32),
        pltpu.VMEM((SCH, 128), jnp.float32),
        pltpu.VMEM((SCH, 128), jnp.float32),
        pltpu.VMEM((6 * TPT,), jnp.int32),
        pltpu.VMEM((TPT,), jnp.int32),
        pltpu.VMEM((TPT,), jnp.int32),
        pltpu.VMEM((TPT,), jnp.int32),
        pltpu.VMEM((TPT,), jnp.int32),
        pltpu.SemaphoreType.DMA,
        pltpu.SemaphoreType.DMA,
        pltpu.SemaphoreType.DMA,
        pltpu.SemaphoreType.DMA,
    ],
)
def _sc_spatial(x0f, y0f, wf, hf, tables, out, tab_sh, srows0,
                srows1, sp_idx, x0b, y0b, wb, hb, gsem0, gsem1, wsem0, wsem1):
  cid = lax.axis_index("c")
  sid = lax.axis_index("s")
  b = cid * BPC + sid // 2
  half = sid % 2
  tok0 = b * S + half * TPT

  pltpu.sync_copy(tables.at[pl.ds(sid * 256, 256)],
                  tab_sh.at[pl.ds(sid * 256, 256)])

  l0 = pltpu.async_copy(x0f.at[pl.ds(tok0, TPT)], x0b, gsem0)
  l1 = pltpu.async_copy(y0f.at[pl.ds(tok0, TPT)], y0b, gsem1)
  l2 = pltpu.async_copy(wf.at[pl.ds(tok0, TPT)], wb, wsem0)
  l3 = pltpu.async_copy(hf.at[pl.ds(tok0, TPT)], hb, wsem1)
  l0.wait()
  l1.wait()
  l2.wait()
  l3.wait()
  for j in range(TPT // 16):
    sl = pl.ds(j * 16, 16)
    x0v = x0b[sl]
    y0v = y0b[sl]
    wv = wb[sl]
    hv = hb[sl]
    o = j * 16
    sp_idx[pl.ds(0 * TPT + o, 16)] = x0v
    sp_idx[pl.ds(1 * TPT + o, 16)] = y0v + 1024
    sp_idx[pl.ds(2 * TPT + o, 16)] = x0v + wv
    sp_idx[pl.ds(3 * TPT + o, 16)] = y0v + hv + 1024
    sp_idx[pl.ds(4 * TPT + o, 16)] = hv + 2048
    sp_idx[pl.ds(5 * TPT + o, 16)] = wv + 3072


  sp_out0 = b * OUT_ROWS_PER_B + MAX_SEG + half * TPT
  sbufs = (srows0, srows1)
  gsems = (gsem0, gsem1)
  wsems = (wsem0, wsem1)
  NSP = 6 * NSC
  sgh = [None] * NSP
  swh = [None] * NSP
  def sp_src(i):
    f, c = i // NSC, i % NSC
    return tab_sh.at[sp_idx.at[pl.ds(f * TPT + c * SCH, SCH)]]
  def sp_dst(i):
    f, c = i // NSC, i % NSC
    return out.at[pl.ds(sp_out0 + c * SCH, SCH), pl.ds(f * 128, 128)]
  plsc.subcore_barrier()
  sgh[0] = pltpu.async_copy(sp_src(0), sbufs[0], gsems[0])
  for i in range(NSP):
    sgh[i].wait()
    swh[i] = pltpu.async_copy(sbufs[i % 2], sp_dst(i), wsems[i % 2])
    if i + 1 < NSP:
      if i >= 1:
        swh[i - 1].wait()
      sgh[i + 1] = pltpu.async_copy(sp_src(i + 1), sbufs[(i + 1) % 2],
                                    gsems[(i + 1) % 2])
  swh[NSP - 2].wait()
  swh[NSP - 1].wait()


def _tc_seg_body(seg_ref, wemb_ref, out_ref):
  seg = seg_ref[0, 0, :]
  iota_m = lax.broadcasted_iota(jnp.int32, (MAX_SEG, S), 0)
  mask = (iota_m == seg[None, :]).astype(jnp.float32)
  counts = jnp.sum(mask, axis=1, keepdims=True)
  mask = mask / jnp.maximum(counts, 1.0)
  out_ref[0] = jnp.dot(mask, wemb_ref[0],
                       preferred_element_type=jnp.float32)


_tc_seg = pl.pallas_call(
    _tc_seg_body,
    grid=(B,),
    in_specs=[
        pl.BlockSpec((1, 1, S), lambda b: (b, 0, 0)),
        pl.BlockSpec((1, S, H), lambda b: (b, 0, 0)),
    ],
    out_specs=pl.BlockSpec((1, MAX_SEG, H), lambda b: (b, 0, 0)),
    out_shape=jax.ShapeDtypeStruct((B, MAX_SEG, H), jnp.float32),
)


def kernel(input_ids, segment_ids, bbox_xy, bbox_wh,
           word_table, x_table, y_table, h_table, w_table):
  ids_f = input_ids.astype(jnp.int32).reshape(-1)
  x0f = bbox_xy[:, :, 0].astype(jnp.int32).reshape(-1)
  y0f = bbox_xy[:, :, 1].astype(jnp.int32).reshape(-1)
  wf = bbox_wh[:, :, 0].astype(jnp.int32).reshape(-1)
  hf = bbox_wh[:, :, 1].astype(jnp.int32).reshape(-1)
  tables = jnp.concatenate([x_table, y_table, h_table, w_table], axis=0)
  wemb = _sc_words(ids_f, word_table)
  seg3d = segment_ids.astype(jnp.int32).reshape(B, 1, S)
  seg_out = _tc_seg(seg3d, wemb.reshape(B, S, H))
  out = _sc_spatial(x0f, y0f, wf, hf, tables)
  out = out.reshape(B, OUT_ROWS_PER_B, H)
  return lax.dynamic_update_slice(out, seg_out, (0, 0, 0))


# speedup vs baseline: 4.9073x; 1.0036x over previous
import functools

import jax
import jax.numpy as jnp
from jax import lax
from jax.experimental import pallas as pl
from jax.experimental.pallas import tpu as pltpu
from jax.experimental.pallas import tpu_sc as plsc

B = 16
S = 768
H = 768
MAX_SEG = 256
OUT_ROWS_PER_B = MAX_SEG + S
NC, NS = 2, 16
BPC = B // NC
TPT = S // 2
WCH = 64
NWC = TPT // WCH
SCH = 128
NSC = TPT // SCH

_mesh = plsc.VectorSubcoreMesh(core_axis_name="c", subcore_axis_name="s")


@functools.partial(
    pl.kernel,
    mesh=_mesh,
    out_type=jax.ShapeDtypeStruct((B * S, H), jnp.float32),
    scratch_types=[
        pltpu.VMEM((WCH, H), jnp.float32),
        pltpu.VMEM((WCH, H), jnp.float32),
        pltpu.VMEM((TPT,), jnp.int32),
        pltpu.SemaphoreType.DMA,
        pltpu.SemaphoreType.DMA,
        pltpu.SemaphoreType.DMA,
        pltpu.SemaphoreType.DMA,
    ],
)
def _sc_words(ids_f, word_table, wemb, wrows0, wrows1, wid_idx,
              gsem0, gsem1, wsem0, wsem1):
  cid = lax.axis_index("c")
  sid = lax.axis_index("s")
  b = cid * BPC + sid // 2
  half = sid % 2
  tok0 = b * S + half * TPT
  pltpu.sync_copy(ids_f.at[pl.ds(tok0, TPT)], wid_idx)

  wbufs = (wrows0, wrows1)
  gsems = (gsem0, gsem1)
  wsems = (wsem0, wsem1)
  gh = [None] * NWC
  wh = [None] * NWC
  gh[0] = pltpu.async_copy(
      word_table.at[wid_idx.at[pl.ds(0, WCH)]], wbufs[0], gsems[0])
  for c in range(NWC):
    gh[c].wait()
    wh[c] = pltpu.async_copy(
        wbufs[c % 2], wemb.at[pl.ds(tok0 + c * WCH, WCH)], wsems[c % 2])
    if c + 1 < NWC:
      if c >= 1:
        wh[c - 1].wait()
      gh[c + 1] = pltpu.async_copy(
          word_table.at[wid_idx.at[pl.ds((c + 1) * WCH, WCH)]],
          wbufs[(c + 1) % 2], gsems[(c + 1) % 2])
  wh[NWC - 2].wait()
  wh[NWC - 1].wait()


@functools.partial(
    pl.kernel,
    mesh=_mesh,
    out_type=jax.ShapeDtypeStruct((B * OUT_ROWS_PER_B, H), jnp.float32),
    scratch_types=[
        pltpu.VMEM_SHARED((4096, 128), jnp.float32),
        pltpu.VMEM((SCH, 128), jnp.float32),
        pltpu.VMEM((SCH, 128), jnp.float32),
        pltpu.VMEM((6 * TPT,), jnp.int32),
        pltpu.VMEM((TPT,), jnp.int32),
        pltpu.VMEM((TPT,), jnp.int32),
        pltpu.VMEM((TPT,), jnp.int32),
        pltpu.VMEM((TPT,), jnp.int32),
        pltpu.SemaphoreType.DMA,
        pltpu.SemaphoreType.DMA,
        pltpu.SemaphoreType.DMA,
        pltpu.SemaphoreType.DMA,
    ],
)
def _sc_spatial(x0f, y0f, wf, hf, tables, out, tab_sh, srows0,
                srows1, sp_idx, x0b, y0b, wb, hb, gsem0, gsem1, wsem0, wsem1):
  cid = lax.axis_index("c")
  sid = lax.axis_index("s")
  b = cid * BPC + sid // 2
  half = sid % 2
  tok0 = b * S + half * TPT

  pltpu.sync_copy(tables.at[pl.ds(sid * 256, 256)],
                  tab_sh.at[pl.ds(sid * 256, 256)])

  l0 = pltpu.async_copy(x0f.at[pl.ds(tok0, TPT)], x0b, gsem0)
  l1 = pltpu.async_copy(y0f.at[pl.ds(tok0, TPT)], y0b, gsem1)
  l2 = pltpu.async_copy(wf.at[pl.ds(tok0, TPT)], wb, wsem0)
  l3 = pltpu.async_copy(hf.at[pl.ds(tok0, TPT)], hb, wsem1)
  l0.wait()
  l1.wait()
  l2.wait()
  l3.wait()
  for j in range(TPT // 16):
    sl = pl.ds(j * 16, 16)
    x0v = x0b[sl]
    y0v = y0b[sl]
    wv = wb[sl]
    hv = hb[sl]
    o = j * 16
    sp_idx[pl.ds(0 * TPT + o, 16)] = x0v
    sp_idx[pl.ds(1 * TPT + o, 16)] = y0v + 1024
    sp_idx[pl.ds(2 * TPT + o, 16)] = x0v + wv
    sp_idx[pl.ds(3 * TPT + o, 16)] = y0v + hv + 1024
    sp_idx[pl.ds(4 * TPT + o, 16)] = hv + 2048
    sp_idx[pl.ds(5 * TPT + o, 16)] = wv + 3072


  sp_out0 = b * OUT_ROWS_PER_B + MAX_SEG + half * TPT
  sbufs = (srows0, srows1)
  gsems = (gsem0, gsem1)
  wsems = (wsem0, wsem1)
  NSP = 6 * NSC
  sgh = [None] * NSP
  swh = [None] * NSP
  def sp_src(i):
    f, c = i // NSC, i % NSC
    return tab_sh.at[sp_idx.at[pl.ds(f * TPT + c * SCH, SCH)]]
  def sp_dst(i):
    f, c = i // NSC, i % NSC
    return out.at[pl.ds(sp_out0 + c * SCH, SCH), pl.ds(f * 128, 128)]
  plsc.subcore_barrier()
  sgh[0] = pltpu.async_copy(sp_src(0), sbufs[0], gsems[0])
  for i in range(NSP):
    sgh[i].wait()
    swh[i] = pltpu.async_copy(sbufs[i % 2], sp_dst(i), wsems[i % 2])
    if i + 1 < NSP:
      if i >= 1:
        swh[i - 1].wait()
      sgh[i + 1] = pltpu.async_copy(sp_src(i + 1), sbufs[(i + 1) % 2],
                                    gsems[(i + 1) % 2])
  swh[NSP - 2].wait()
  swh[NSP - 1].wait()


def _tc_seg_body(seg_ref, wemb_ref, out_ref):
  seg = seg_ref[0, 0, :]
  iota_m = lax.broadcasted_iota(jnp.int32, (MAX_SEG, S), 0)
  mask = (iota_m == seg[None, :]).astype(jnp.float32)
  counts = jnp.sum(mask, axis=1, keepdims=True)
  mask = (mask / jnp.maximum(counts, 1.0)).astype(jnp.bfloat16)
  out_ref[0] = jnp.dot(mask, wemb_ref[0].astype(jnp.bfloat16),
                       preferred_element_type=jnp.float32)


_tc_seg = pl.pallas_call(
    _tc_seg_body,
    grid=(B,),
    in_specs=[
        pl.BlockSpec((1, 1, S), lambda b: (b, 0, 0)),
        pl.BlockSpec((1, S, H), lambda b: (b, 0, 0)),
    ],
    out_specs=pl.BlockSpec((1, MAX_SEG, H), lambda b: (b, 0, 0)),
    out_shape=jax.ShapeDtypeStruct((B, MAX_SEG, H), jnp.float32),
)


def kernel(input_ids, segment_ids, bbox_xy, bbox_wh,
           word_table, x_table, y_table, h_table, w_table):
  ids_f = input_ids.astype(jnp.int32).reshape(-1)
  x0f = bbox_xy[:, :, 0].astype(jnp.int32).reshape(-1)
  y0f = bbox_xy[:, :, 1].astype(jnp.int32).reshape(-1)
  wf = bbox_wh[:, :, 0].astype(jnp.int32).reshape(-1)
  hf = bbox_wh[:, :, 1].astype(jnp.int32).reshape(-1)
  tables = jnp.concatenate([x_table, y_table, h_table, w_table], axis=0)
  wemb = _sc_words(ids_f, word_table)
  seg3d = segment_ids.astype(jnp.int32).reshape(B, 1, S)
  seg_out = _tc_seg(seg3d, wemb.reshape(B, S, H))
  out = _sc_spatial(x0f, y0f, wf, hf, tables)
  out = out.reshape(B, OUT_ROWS_PER_B, H)
  return lax.dynamic_update_slice(out, seg_out, (0, 0, 0))
